# baseline (speedup 1.0000x reference)
import jax
import jax.numpy as jnp
from jax import lax
from jax.experimental import pallas as pl
from jax.experimental.pallas import tpu as pltpu
from jax.experimental.pallas import tpu_sc as plsc

H, W = 1536, 2048
K = 5000
NB = 4096
EDGE_T = (10 + 1) ** 2 / 10.0


def _stencil_body(top_ref, mid_ref, bot_ref, out_ref):
    i = pl.program_id(0)
    s = jnp.concatenate([top_ref[...], mid_ref[...], bot_ref[...]], axis=0)
    sm = s[8:136]
    l1 = jnp.roll(s, 1, axis=1)
    r1 = jnp.roll(s, -1, axis=1)
    rowmax = jnp.maximum(jnp.maximum(l1, s), r1)
    m9 = jnp.maximum(jnp.maximum(rowmax[7:135], rowmax[8:136]), rowmax[9:137])
    nms = jnp.logical_and(sm > 0.5, sm == m9)
    sb = s.astype(jnp.bfloat16).astype(jnp.float32)
    smb = sb[8:136]
    l3 = jnp.roll(sb, 3, axis=1)
    r3 = jnp.roll(sb, -3, axis=1)
    dii = sb[5:133] - 2.0 * smb + sb[11:139]
    djj = l3[8:136] - 2.0 * smb + r3[8:136]
    dij = 0.25 * (l3[5:133] - r3[5:133] - l3[11:139] + r3[11:139])
    det = dii * djj - dij * dij
    tr = dii + djj
    edge = jnp.logical_and(tr * tr / det <= EDGE_T, det > 0)
    r = lax.broadcasted_iota(jnp.int32, (128, W), 0) + i * 128
    c = lax.broadcasted_iota(jnp.int32, (128, W), 1)
    eof = (r >= 5) & (r < H - 5) & (c >= 5) & (c < W - 5)
    keep = nms & eof & edge
    out_ref[...] = jnp.where(keep, sm, 0.0)


def _masked_map(score2d):
    return pl.pallas_call(
        _stencil_body,
        grid=(12,),
        in_specs=[
            pl.BlockSpec((8, W), lambda i: (jnp.maximum(16 * i - 1, 0), 0)),
            pl.BlockSpec((128, W), lambda i: (i, 0)),
            pl.BlockSpec((8, W), lambda i: (jnp.minimum(16 * i + 16, 191), 0)),
        ],
        out_specs=pl.BlockSpec((128, W), lambda i: (i, 0)),
        out_shape=jax.ShapeDtypeStruct((H, W), jnp.float32),
    )(score2d, score2d, score2d)


NW = 32
ROWS_W = H // NW
PER_W = ROWS_W * W
CAP_T = 256
L = 16
PK_CAP = 12288
PK_PAD = PK_CAP + W + L
CK_ROWS = 16
N_CK = ROWS_W // CK_ROWS

_sc_mesh = None


def _mesh():
    global _sc_mesh
    if _sc_mesh is None:
        _sc_mesh = plsc.VectorSubcoreMesh(
            core_axis_name="c", subcore_axis_name="s",
            num_cores=2, num_subcores=16)
    return _sc_mesh


def _scan_body(x_hbm, hist_hbm, pks_hbm, pki_hbm,
               buf0, buf1, pks, pki, hist, sem0, sem1):
    wid = lax.axis_index("c") * 16 + lax.axis_index("s")
    base = wid * PER_W
    zerosf = jnp.zeros((L,), jnp.float32)
    zerosi = jnp.zeros((L,), jnp.int32)

    def zh(i, _):
        hist[pl.ds(i * L, L)] = zerosi
        return 0
    lax.fori_loop(0, NB // L, zh, 0)

    def zp(i, _):
        pks[pl.ds(i * L, L)] = zerosf
        pki[pl.ds(i * L, L)] = zerosi
        return 0
    lax.fori_loop(0, PK_PAD // L, zp, 0)

    bufs = (buf0, buf1)
    sems = (sem0, sem1)
    cps = [None, None]
    cps[0] = pltpu.async_copy(
        x_hbm.at[pl.ds(wid * ROWS_W, CK_ROWS)], buf0, sem0)
    lanes = lax.iota(jnp.int32, L)
    off = jnp.int32(0)
    for ci in range(N_CK):
        if ci + 1 < N_CK:
            cps[(ci + 1) % 2] = pltpu.async_copy(
                x_hbm.at[pl.ds(wid * ROWS_W + (ci + 1) * CK_ROWS, CK_ROWS)],
                bufs[(ci + 1) % 2], sems[(ci + 1) % 2])
        cps[ci % 2].wait()
        buf = bufs[ci % 2]

        def rbody(r, off, _ci=ci, _buf=buf):
            rowbase = base + (_ci * CK_ROWS) * W + r * W

            def vbody(j, off):
                v = _buf[r, pl.ds(j * L, L)]
                m = v > 0.5
                iv = (rowbase + j * L) + lanes
                plsc.store_compressed(pks.at[pl.ds(off, L)], v, mask=m)
                plsc.store_compressed(pki.at[pl.ds(off, L)], iv, mask=m)
                return off + plsc.all_reduce_population_count(m)[0]
            off = lax.fori_loop(0, W // L, vbody, off)
            return jnp.minimum(off, PK_CAP)
        off = lax.fori_loop(0, CK_ROWS, rbody, off)
    ones = jnp.ones((L,), jnp.int32)

    def hbody(i, _):
        v = pks[pl.ds(i * L, L)]
        m = v > 0.5
        b = ((v - 0.5) * (2 * NB)).astype(jnp.int32)
        plsc.addupdate_scatter(hist, [b], ones, mask=m)
        return 0
    lax.fori_loop(0, (off + (L - 1)) // L, hbody, 0)
    pltpu.sync_copy(hist, hist_hbm.at[wid])
    pltpu.sync_copy(pks.at[pl.ds(0, PK_CAP)], pks_hbm.at[wid])
    pltpu.sync_copy(pki.at[pl.ds(0, PK_CAP)], pki_hbm.at[wid])


def _sc_scan(masked2d):
    return pl.kernel(
        _scan_body,
        out_type=(
            jax.ShapeDtypeStruct((NW, NB), jnp.int32),
            jax.ShapeDtypeStruct((NW, PK_CAP), jnp.float32),
            jax.ShapeDtypeStruct((NW, PK_CAP), jnp.int32),
        ),
        mesh=_mesh(),
        compiler_params=pltpu.CompilerParams(needs_layout_passes=False),
        scratch_types=[
            pltpu.VMEM((CK_ROWS, W), jnp.float32),
            pltpu.VMEM((CK_ROWS, W), jnp.float32),
            pltpu.VMEM((PK_PAD,), jnp.float32),
            pltpu.VMEM((PK_PAD,), jnp.int32),
            pltpu.VMEM((NB,), jnp.int32),
            pltpu.SemaphoreType.DMA,
            pltpu.SemaphoreType.DMA,
        ],
    )(masked2d)


def _select_body(pks_hbm, pki_hbm, lo_hbm, sc_hbm, ix_hbm,
                 ps, pi, sbuf, ibuf, lvec):
    wid = lax.axis_index("c") * 16 + lax.axis_index("s")
    neg = jnp.full((L,), -jnp.inf, jnp.float32)
    zer = jnp.zeros((L,), jnp.int32)

    def zbody(i, _):
        sbuf[pl.ds(i * L, L)] = neg
        ibuf[pl.ds(i * L, L)] = zer
        return 0
    lax.fori_loop(0, CAP_T // L, zbody, 0)
    pltpu.sync_copy(lo_hbm, lvec)
    lo = lvec[...]
    pltpu.sync_copy(pks_hbm.at[wid], ps)
    pltpu.sync_copy(pki_hbm.at[wid], pi)

    def vbody(i, cnt):
        sl = pl.ds(i * L, L)
        v = ps[sl]
        m = v >= lo
        mi = m.astype(jnp.int32)
        pos = cnt + plsc.cumsum(mi) - mi
        mg = jnp.logical_and(m, pos < CAP_T)
        posc = jnp.minimum(pos, CAP_T - 1)
        plsc.store_scatter(sbuf, [posc], v, mask=mg)
        plsc.store_scatter(ibuf, [posc], pi[sl], mask=mg)
        return cnt + jnp.sum(mi)
    lax.fori_loop(0, PK_CAP // L, vbody, jnp.int32(0))
    pltpu.sync_copy(sbuf, sc_hbm.at[pl.ds(wid * CAP_T, CAP_T)])
    pltpu.sync_copy(ibuf, ix_hbm.at[pl.ds(wid * CAP_T, CAP_T)])


def _sc_select(pks, pki, lo_vec):
    return pl.kernel(
        _select_body,
        out_type=(
            jax.ShapeDtypeStruct((NW * CAP_T,), jnp.float32),
            jax.ShapeDtypeStruct((NW * CAP_T,), jnp.int32),
        ),
        mesh=_mesh(),
        compiler_params=pltpu.CompilerParams(needs_layout_passes=False),
        scratch_types=[
            pltpu.VMEM((PK_CAP,), jnp.float32),
            pltpu.VMEM((PK_CAP,), jnp.int32),
            pltpu.VMEM((CAP_T,), jnp.float32),
            pltpu.VMEM((CAP_T,), jnp.int32),
            pltpu.VMEM((L,), jnp.float32),
        ],
    )(pks, pki, lo_vec)


def kernel(score_map, k):
    score2d = score_map.reshape(H, W)
    masked = _masked_map(score2d)
    hist32, pks, pki = _sc_scan(masked)
    hist = hist32.sum(axis=0)
    above = jnp.cumsum(hist[::-1])[::-1] >= K
    bstar = jnp.maximum(jnp.sum(above.astype(jnp.int32)) - 1, 0)
    lo = 0.5 + bstar.astype(jnp.float32) * (1.0 / (2 * NB))
    lo_vec = jnp.full((L,), lo, jnp.float32)
    cscores, cidx = _sc_select(pks, pki, lo_vec)
    negs, topi = lax.sort_key_val(-cscores, cidx)
    topv = -negs[:K]
    topi = topi[:K]
    nfin = jnp.sum((topv > 0.5).astype(jnp.int32))
    filler = lax.iota(jnp.int32, K) - nfin
    topi = jnp.where(topv > 0.5, topi, filler)
    topv = jnp.where(topv > 0.5, topv, -jnp.inf)
    topi = topi + jnp.asarray(k - k, dtype=topi.dtype)
    indices = jnp.stack([topi // W, topi % W], axis=-1)
    return (indices[None], topv[None])


# speedup vs baseline: 1.2008x; 1.2008x over previous
import jax
import jax.numpy as jnp
from jax import lax
from jax.experimental import pallas as pl
from jax.experimental.pallas import tpu as pltpu
from jax.experimental.pallas import tpu_sc as plsc

H, W = 1536, 2048
K = 5000
NB = 4096
EDGE_T = (10 + 1) ** 2 / 10.0


def _stencil_body(top_ref, mid_ref, bot_ref, out_ref):
    i = pl.program_id(0)
    s = jnp.concatenate([top_ref[...], mid_ref[...], bot_ref[...]], axis=0)
    sm = s[8:136]
    l1 = jnp.roll(s, 1, axis=1)
    r1 = jnp.roll(s, -1, axis=1)
    rowmax = jnp.maximum(jnp.maximum(l1, s), r1)
    m9 = jnp.maximum(jnp.maximum(rowmax[7:135], rowmax[8:136]), rowmax[9:137])
    nms = jnp.logical_and(sm > 0.5, sm == m9)
    sb = s.astype(jnp.bfloat16).astype(jnp.float32)
    smb = sb[8:136]
    l3 = jnp.roll(sb, 3, axis=1)
    r3 = jnp.roll(sb, -3, axis=1)
    dii = sb[5:133] - 2.0 * smb + sb[11:139]
    djj = l3[8:136] - 2.0 * smb + r3[8:136]
    dij = 0.25 * (l3[5:133] - r3[5:133] - l3[11:139] + r3[11:139])
    det = dii * djj - dij * dij
    tr = dii + djj
    edge = jnp.logical_and(tr * tr / det <= EDGE_T, det > 0)
    r = lax.broadcasted_iota(jnp.int32, (128, W), 0) + i * 128
    c = lax.broadcasted_iota(jnp.int32, (128, W), 1)
    eof = (r >= 5) & (r < H - 5) & (c >= 5) & (c < W - 5)
    keep = nms & eof & edge
    out_ref[...] = jnp.where(keep, sm, 0.0)


def _masked_map(score2d):
    return pl.pallas_call(
        _stencil_body,
        grid=(12,),
        in_specs=[
            pl.BlockSpec((8, W), lambda i: (jnp.maximum(16 * i - 1, 0), 0)),
            pl.BlockSpec((128, W), lambda i: (i, 0)),
            pl.BlockSpec((8, W), lambda i: (jnp.minimum(16 * i + 16, 191), 0)),
        ],
        out_specs=pl.BlockSpec((128, W), lambda i: (i, 0)),
        out_shape=jax.ShapeDtypeStruct((H, W), jnp.float32),
    )(score2d, score2d, score2d)


NW = 32
ROWS_W = H // NW
PER_W = ROWS_W * W
CAP_T = 256
L = 16
PK_CAP = 12288
PK_PAD = PK_CAP + W + L
CK_ROWS = 16
N_CK = ROWS_W // CK_ROWS

_sc_mesh = None


def _mesh():
    global _sc_mesh
    if _sc_mesh is None:
        _sc_mesh = plsc.VectorSubcoreMesh(
            core_axis_name="c", subcore_axis_name="s",
            num_cores=2, num_subcores=16)
    return _sc_mesh


def _scan_body(x_hbm, hist_hbm, pks_hbm, pki_hbm,
               buf0, buf1, pks, pki, hist, sem0, sem1):
    wid = lax.axis_index("c") * 16 + lax.axis_index("s")
    base = wid * PER_W
    zerosf = jnp.zeros((L,), jnp.float32)
    zerosi = jnp.zeros((L,), jnp.int32)

    def zh(i, _):
        hist[pl.ds(i * L, L)] = zerosi
        return 0
    lax.fori_loop(0, NB // L, zh, 0)

    def zp(i, _):
        pks[pl.ds(i * L, L)] = zerosf
        pki[pl.ds(i * L, L)] = zerosi
        return 0
    lax.fori_loop(0, PK_PAD // L, zp, 0)

    bufs = (buf0, buf1)
    sems = (sem0, sem1)
    cps = [None, None]
    cps[0] = pltpu.async_copy(
        x_hbm.at[pl.ds(wid * ROWS_W, CK_ROWS)], buf0, sem0)
    lanes = lax.iota(jnp.int32, L)
    off = jnp.int32(0)
    for ci in range(N_CK):
        if ci + 1 < N_CK:
            cps[(ci + 1) % 2] = pltpu.async_copy(
                x_hbm.at[pl.ds(wid * ROWS_W + (ci + 1) * CK_ROWS, CK_ROWS)],
                bufs[(ci + 1) % 2], sems[(ci + 1) % 2])
        cps[ci % 2].wait()
        buf = bufs[ci % 2]

        def rbody(r, off, _ci=ci, _buf=buf):
            rowbase = base + (_ci * CK_ROWS) * W + r * W

            def vbody(j, off):
                vs, ms, cs = [], [], []
                for u in range(4):
                    v = _buf[r, pl.ds((j * 4 + u) * L, L)]
                    m = v > 0.5
                    vs.append(v)
                    ms.append(m)
                    cs.append(jnp.sum(m.astype(jnp.int32)))
                for u in range(4):
                    iv = (rowbase + (j * 4 + u) * L) + lanes
                    plsc.store_compressed(pks.at[pl.ds(off, L)], vs[u], mask=ms[u])
                    plsc.store_compressed(pki.at[pl.ds(off, L)], iv, mask=ms[u])
                    off = off + cs[u]
                return off
            off = lax.fori_loop(0, W // L // 4, vbody, off)
            return jnp.minimum(off, PK_CAP)
        off = lax.fori_loop(0, CK_ROWS, rbody, off)
    ones = jnp.ones((L,), jnp.int32)

    def hbody(i, _):
        v = pks[pl.ds(i * L, L)]
        m = v > 0.5
        b = ((v - 0.5) * (2 * NB)).astype(jnp.int32)
        plsc.addupdate_scatter(hist, [b], ones, mask=m)
        return 0
    lax.fori_loop(0, (off + (L - 1)) // L, hbody, 0)
    pltpu.sync_copy(hist, hist_hbm.at[wid])
    pltpu.sync_copy(pks.at[pl.ds(0, PK_CAP)], pks_hbm.at[wid])
    pltpu.sync_copy(pki.at[pl.ds(0, PK_CAP)], pki_hbm.at[wid])


def _sc_scan(masked2d):
    return pl.kernel(
        _scan_body,
        out_type=(
            jax.ShapeDtypeStruct((NW, NB), jnp.int32),
            jax.ShapeDtypeStruct((NW, PK_CAP), jnp.float32),
            jax.ShapeDtypeStruct((NW, PK_CAP), jnp.int32),
        ),
        mesh=_mesh(),
        compiler_params=pltpu.CompilerParams(needs_layout_passes=False),
        scratch_types=[
            pltpu.VMEM((CK_ROWS, W), jnp.float32),
            pltpu.VMEM((CK_ROWS, W), jnp.float32),
            pltpu.VMEM((PK_PAD,), jnp.float32),
            pltpu.VMEM((PK_PAD,), jnp.int32),
            pltpu.VMEM((NB,), jnp.int32),
            pltpu.SemaphoreType.DMA,
            pltpu.SemaphoreType.DMA,
        ],
    )(masked2d)


def _select_body(pks_hbm, pki_hbm, lo_hbm, sc_hbm, ix_hbm,
                 ps, pi, sbuf, ibuf, lvec):
    wid = lax.axis_index("c") * 16 + lax.axis_index("s")
    neg = jnp.full((L,), -jnp.inf, jnp.float32)
    zer = jnp.zeros((L,), jnp.int32)

    def zbody(i, _):
        sbuf[pl.ds(i * L, L)] = neg
        ibuf[pl.ds(i * L, L)] = zer
        return 0
    lax.fori_loop(0, CAP_T // L, zbody, 0)
    pltpu.sync_copy(lo_hbm, lvec)
    lo = lvec[...]
    pltpu.sync_copy(pks_hbm.at[wid], ps)
    pltpu.sync_copy(pki_hbm.at[wid], pi)

    def vbody(i, cnt):
        sl = pl.ds(i * L, L)
        v = ps[sl]
        m = v >= lo
        mi = m.astype(jnp.int32)
        pos = cnt + plsc.cumsum(mi) - mi
        mg = jnp.logical_and(m, pos < CAP_T)
        posc = jnp.minimum(pos, CAP_T - 1)
        plsc.store_scatter(sbuf, [posc], v, mask=mg)
        plsc.store_scatter(ibuf, [posc], pi[sl], mask=mg)
        return cnt + jnp.sum(mi)
    lax.fori_loop(0, PK_CAP // L, vbody, jnp.int32(0))
    pltpu.sync_copy(sbuf, sc_hbm.at[pl.ds(wid * CAP_T, CAP_T)])
    pltpu.sync_copy(ibuf, ix_hbm.at[pl.ds(wid * CAP_T, CAP_T)])


def _sc_select(pks, pki, lo_vec):
    return pl.kernel(
        _select_body,
        out_type=(
            jax.ShapeDtypeStruct((NW * CAP_T,), jnp.float32),
            jax.ShapeDtypeStruct((NW * CAP_T,), jnp.int32),
        ),
        mesh=_mesh(),
        compiler_params=pltpu.CompilerParams(needs_layout_passes=False),
        scratch_types=[
            pltpu.VMEM((PK_CAP,), jnp.float32),
            pltpu.VMEM((PK_CAP,), jnp.int32),
            pltpu.VMEM((CAP_T,), jnp.float32),
            pltpu.VMEM((CAP_T,), jnp.int32),
            pltpu.VMEM((L,), jnp.float32),
        ],
    )(pks, pki, lo_vec)


def kernel(score_map, k):
    score2d = score_map.reshape(H, W)
    masked = _masked_map(score2d)
    hist32, pks, pki = _sc_scan(masked)
    hist = hist32.sum(axis=0)
    above = jnp.cumsum(hist[::-1])[::-1] >= K
    bstar = jnp.maximum(jnp.sum(above.astype(jnp.int32)) - 1, 0)
    lo = 0.5 + bstar.astype(jnp.float32) * (1.0 / (2 * NB))
    lo_vec = jnp.full((L,), lo, jnp.float32)
    cscores, cidx = _sc_select(pks, pki, lo_vec)
    negs, topi = lax.sort_key_val(-cscores, cidx)
    topv = -negs[:K]
    topi = topi[:K]
    nfin = jnp.sum((topv > 0.5).astype(jnp.int32))
    filler = lax.iota(jnp.int32, K) - nfin
    topi = jnp.where(topv > 0.5, topi, filler)
    topv = jnp.where(topv > 0.5, topv, -jnp.inf)
    topi = topi + jnp.asarray(k - k, dtype=topi.dtype)
    indices = jnp.stack([topi // W, topi % W], axis=-1)
    return (indices[None], topv[None])


# speedup vs baseline: 1.4415x; 1.2005x over previous
import jax
import jax.numpy as jnp
from jax import lax
from jax.experimental import pallas as pl
from jax.experimental.pallas import tpu as pltpu
from jax.experimental.pallas import tpu_sc as plsc

H, W = 1536, 2048
K = 5000
NB = 4096
EDGE_T = (10 + 1) ** 2 / 10.0


def _stencil_body(top_ref, mid_ref, bot_ref, out_ref):
    i = pl.program_id(0)
    s = jnp.concatenate([top_ref[...], mid_ref[...], bot_ref[...]], axis=0)
    sm = s[8:136]
    l1 = jnp.roll(s, 1, axis=1)
    r1 = jnp.roll(s, -1, axis=1)
    rowmax = jnp.maximum(jnp.maximum(l1, s), r1)
    m9 = jnp.maximum(jnp.maximum(rowmax[7:135], rowmax[8:136]), rowmax[9:137])
    nms = jnp.logical_and(sm > 0.5, sm == m9)
    sb = s.astype(jnp.bfloat16).astype(jnp.float32)
    smb = sb[8:136]
    l3 = jnp.roll(sb, 3, axis=1)
    r3 = jnp.roll(sb, -3, axis=1)
    dii = sb[5:133] - 2.0 * smb + sb[11:139]
    djj = l3[8:136] - 2.0 * smb + r3[8:136]
    dij = 0.25 * (l3[5:133] - r3[5:133] - l3[11:139] + r3[11:139])
    det = dii * djj - dij * dij
    tr = dii + djj
    edge = jnp.logical_and(tr * tr / det <= EDGE_T, det > 0)
    r = lax.broadcasted_iota(jnp.int32, (128, W), 0) + i * 128
    c = lax.broadcasted_iota(jnp.int32, (128, W), 1)
    eof = (r >= 5) & (r < H - 5) & (c >= 5) & (c < W - 5)
    keep = nms & eof & edge
    out_ref[...] = jnp.where(keep, sm, 0.0)


def _masked_map(score2d):
    return pl.pallas_call(
        _stencil_body,
        grid=(12,),
        in_specs=[
            pl.BlockSpec((8, W), lambda i: (jnp.maximum(16 * i - 1, 0), 0)),
            pl.BlockSpec((128, W), lambda i: (i, 0)),
            pl.BlockSpec((8, W), lambda i: (jnp.minimum(16 * i + 16, 191), 0)),
        ],
        out_specs=pl.BlockSpec((128, W), lambda i: (i, 0)),
        out_shape=jax.ShapeDtypeStruct((H, W), jnp.float32),
    )(score2d, score2d, score2d)


NW = 32
ROWS_W = H // NW
PER_W = ROWS_W * W
CAP_T = 256
L = 16
PK_CAP = 12288
PK_PAD = PK_CAP + W + L
CK_ROWS = 16
N_CK = ROWS_W // CK_ROWS

_sc_mesh = None


def _mesh():
    global _sc_mesh
    if _sc_mesh is None:
        _sc_mesh = plsc.VectorSubcoreMesh(
            core_axis_name="c", subcore_axis_name="s",
            num_cores=2, num_subcores=16)
    return _sc_mesh


def _scan_body(x_hbm, hist_hbm, pks_hbm, pki_hbm,
               buf0, buf1, pks, pki, hist, sem0, sem1):
    wid = lax.axis_index("c") * 16 + lax.axis_index("s")
    base = wid * PER_W
    zerosf = jnp.zeros((L,), jnp.float32)
    zerosi = jnp.zeros((L,), jnp.int32)

    def zh(i, _):
        hist[pl.ds(i * L, L)] = zerosi
        return 0
    lax.fori_loop(0, NB // L, zh, 0)

    def zp(i, _):
        pks[pl.ds(i * L, L)] = zerosf
        pki[pl.ds(i * L, L)] = zerosi
        return 0
    lax.fori_loop(0, PK_PAD // L, zp, 0)

    bufs = (buf0, buf1)
    sems = (sem0, sem1)
    cps = [None, None]
    cps[0] = pltpu.async_copy(
        x_hbm.at[pl.ds(wid * ROWS_W, CK_ROWS)], buf0, sem0)
    lanes = lax.iota(jnp.int32, L)
    off = jnp.int32(0)
    for ci in range(N_CK):
        if ci + 1 < N_CK:
            cps[(ci + 1) % 2] = pltpu.async_copy(
                x_hbm.at[pl.ds(wid * ROWS_W + (ci + 1) * CK_ROWS, CK_ROWS)],
                bufs[(ci + 1) % 2], sems[(ci + 1) % 2])
        cps[ci % 2].wait()
        buf = bufs[ci % 2]

        def rbody(r, off, _ci=ci, _buf=buf):
            rowbase = base + (_ci * CK_ROWS) * W + r * W

            def vbody(j, off):
                vs, ms, cs = [], [], []
                for u in range(8):
                    v = _buf[r, pl.ds((j * 8 + u) * L, L)]
                    m = v > 0.5
                    vs.append(v)
                    ms.append(m)
                    cs.append(jnp.sum(m.astype(jnp.int32)))
                for u in range(8):
                    iv = (rowbase + (j * 8 + u) * L) + lanes
                    plsc.store_compressed(pks.at[pl.ds(off, L)], vs[u], mask=ms[u])
                    plsc.store_compressed(pki.at[pl.ds(off, L)], iv, mask=ms[u])
                    off = off + cs[u]
                return off
            off = lax.fori_loop(0, W // L // 8, vbody, off)
            return jnp.minimum(off, PK_CAP)
        off = lax.fori_loop(0, CK_ROWS, rbody, off)
    ones = jnp.ones((L,), jnp.int32)

    def hbody(i, _):
        v = pks[pl.ds(i * L, L)]
        m = v > 0.5
        b = ((v - 0.5) * (2 * NB)).astype(jnp.int32)
        plsc.addupdate_scatter(hist, [b], ones, mask=m)
        return 0
    lax.fori_loop(0, (off + (L - 1)) // L, hbody, 0)
    pltpu.sync_copy(hist, hist_hbm.at[wid])
    pltpu.sync_copy(pks.at[pl.ds(0, PK_CAP)], pks_hbm.at[wid])
    pltpu.sync_copy(pki.at[pl.ds(0, PK_CAP)], pki_hbm.at[wid])


def _sc_scan(masked2d):
    return pl.kernel(
        _scan_body,
        out_type=(
            jax.ShapeDtypeStruct((NW, NB), jnp.int32),
            jax.ShapeDtypeStruct((NW, PK_CAP), jnp.float32),
            jax.ShapeDtypeStruct((NW, PK_CAP), jnp.int32),
        ),
        mesh=_mesh(),
        compiler_params=pltpu.CompilerParams(needs_layout_passes=False),
        scratch_types=[
            pltpu.VMEM((CK_ROWS, W), jnp.float32),
            pltpu.VMEM((CK_ROWS, W), jnp.float32),
            pltpu.VMEM((PK_PAD,), jnp.float32),
            pltpu.VMEM((PK_PAD,), jnp.int32),
            pltpu.VMEM((NB,), jnp.int32),
            pltpu.SemaphoreType.DMA,
            pltpu.SemaphoreType.DMA,
        ],
    )(masked2d)


def _select_body(pks_hbm, pki_hbm, lo_hbm, sc_hbm, ix_hbm,
                 ps, pi, sbuf, ibuf, lvec):
    wid = lax.axis_index("c") * 16 + lax.axis_index("s")
    neg = jnp.full((L,), -jnp.inf, jnp.float32)
    zer = jnp.zeros((L,), jnp.int32)

    def zbody(i, _):
        sbuf[pl.ds(i * L, L)] = neg
        ibuf[pl.ds(i * L, L)] = zer
        return 0
    lax.fori_loop(0, CAP_T // L, zbody, 0)
    pltpu.sync_copy(lo_hbm, lvec)
    lo = lvec[...]
    pltpu.sync_copy(pks_hbm.at[wid], ps)
    pltpu.sync_copy(pki_hbm.at[wid], pi)

    def vbody(i, cnt):
        vs, ms, exs, cs = [], [], [], []
        for u in range(4):
            sl = pl.ds((i * 4 + u) * L, L)
            v = ps[sl]
            m = v >= lo
            mi = m.astype(jnp.int32)
            vs.append(v)
            ms.append(m)
            exs.append(plsc.cumsum(mi) - mi)
            cs.append(jnp.sum(mi))
        for u in range(4):
            pos = cnt + exs[u]
            mg = jnp.logical_and(ms[u], pos < CAP_T)
            posc = jnp.minimum(pos, CAP_T - 1)
            plsc.store_scatter(sbuf, [posc], vs[u], mask=mg)
            plsc.store_scatter(ibuf, [posc], pi[pl.ds((i * 4 + u) * L, L)],
                               mask=mg)
            cnt = cnt + cs[u]
        return cnt
    lax.fori_loop(0, PK_CAP // L // 4, vbody, jnp.int32(0))
    pltpu.sync_copy(sbuf, sc_hbm.at[pl.ds(wid * CAP_T, CAP_T)])
    pltpu.sync_copy(ibuf, ix_hbm.at[pl.ds(wid * CAP_T, CAP_T)])


def _sc_select(pks, pki, lo_vec):
    return pl.kernel(
        _select_body,
        out_type=(
            jax.ShapeDtypeStruct((NW * CAP_T,), jnp.float32),
            jax.ShapeDtypeStruct((NW * CAP_T,), jnp.int32),
        ),
        mesh=_mesh(),
        compiler_params=pltpu.CompilerParams(needs_layout_passes=False),
        scratch_types=[
            pltpu.VMEM((PK_CAP,), jnp.float32),
            pltpu.VMEM((PK_CAP,), jnp.int32),
            pltpu.VMEM((CAP_T,), jnp.float32),
            pltpu.VMEM((CAP_T,), jnp.int32),
            pltpu.VMEM((L,), jnp.float32),
        ],
    )(pks, pki, lo_vec)


def kernel(score_map, k):
    score2d = score_map.reshape(H, W)
    masked = _masked_map(score2d)
    hist32, pks, pki = _sc_scan(masked)
    hist = hist32.sum(axis=0)
    above = jnp.cumsum(hist[::-1])[::-1] >= K
    bstar = jnp.maximum(jnp.sum(above.astype(jnp.int32)) - 1, 0)
    lo = 0.5 + bstar.astype(jnp.float32) * (1.0 / (2 * NB))
    lo_vec = jnp.full((L,), lo, jnp.float32)
    cscores, cidx = _sc_select(pks, pki, lo_vec)
    negs, topi = lax.sort_key_val(-cscores, cidx)
    topv = -negs[:K]
    topi = topi[:K]
    nfin = jnp.sum((topv > 0.5).astype(jnp.int32))
    filler = lax.iota(jnp.int32, K) - nfin
    topi = jnp.where(topv > 0.5, topi, filler)
    topv = jnp.where(topv > 0.5, topv, -jnp.inf)
    topi = topi + jnp.asarray(k - k, dtype=topi.dtype)
    indices = jnp.stack([topi // W, topi % W], axis=-1)
    return (indices[None], topv[None])


# speedup vs baseline: 1.5235x; 1.0569x over previous
import jax
import jax.numpy as jnp
from jax import lax
from jax.experimental import pallas as pl
from jax.experimental.pallas import tpu as pltpu
from jax.experimental.pallas import tpu_sc as plsc

H, W = 1536, 2048
K = 5000
NB = 4096
EDGE_T = (10 + 1) ** 2 / 10.0


def _stencil_body(top_ref, mid_ref, bot_ref, out_ref):
    i = pl.program_id(0)
    s = jnp.concatenate([top_ref[...], mid_ref[...], bot_ref[...]], axis=0)
    sm = s[8:136]
    l1 = jnp.roll(s, 1, axis=1)
    r1 = jnp.roll(s, -1, axis=1)
    rowmax = jnp.maximum(jnp.maximum(l1, s), r1)
    m9 = jnp.maximum(jnp.maximum(rowmax[7:135], rowmax[8:136]), rowmax[9:137])
    nms = jnp.logical_and(sm > 0.5, sm == m9)
    sb = s.astype(jnp.bfloat16).astype(jnp.float32)
    smb = sb[8:136]
    l3 = jnp.roll(sb, 3, axis=1)
    r3 = jnp.roll(sb, -3, axis=1)
    dii = sb[5:133] - 2.0 * smb + sb[11:139]
    djj = l3[8:136] - 2.0 * smb + r3[8:136]
    dij = 0.25 * (l3[5:133] - r3[5:133] - l3[11:139] + r3[11:139])
    det = dii * djj - dij * dij
    tr = dii + djj
    edge = jnp.logical_and(tr * tr / det <= EDGE_T, det > 0)
    r = lax.broadcasted_iota(jnp.int32, (128, W), 0) + i * 128
    c = lax.broadcasted_iota(jnp.int32, (128, W), 1)
    eof = (r >= 5) & (r < H - 5) & (c >= 5) & (c < W - 5)
    keep = nms & eof & edge
    out_ref[...] = jnp.where(keep, sm, 0.0)


def _masked_map(score2d):
    return pl.pallas_call(
        _stencil_body,
        grid=(12,),
        in_specs=[
            pl.BlockSpec((8, W), lambda i: (jnp.maximum(16 * i - 1, 0), 0)),
            pl.BlockSpec((128, W), lambda i: (i, 0)),
            pl.BlockSpec((8, W), lambda i: (jnp.minimum(16 * i + 16, 191), 0)),
        ],
        out_specs=pl.BlockSpec((128, W), lambda i: (i, 0)),
        out_shape=jax.ShapeDtypeStruct((H, W), jnp.float32),
    )(score2d, score2d, score2d)


NW = 32
ROWS_W = H // NW
PER_W = ROWS_W * W
CAP_T = 256
L = 16
PK_CAP = 12288
PK_PAD = PK_CAP + W + L
CK_ROWS = 16
N_CK = ROWS_W // CK_ROWS

_sc_mesh = None


def _mesh():
    global _sc_mesh
    if _sc_mesh is None:
        _sc_mesh = plsc.VectorSubcoreMesh(
            core_axis_name="c", subcore_axis_name="s",
            num_cores=2, num_subcores=16)
    return _sc_mesh


def _scan_body(x_hbm, hist_hbm, pks_hbm, pki_hbm,
               buf0, buf1, pks, pki, hist, sem0, sem1):
    wid = lax.axis_index("c") * 16 + lax.axis_index("s")
    base = wid * PER_W
    zerosf = jnp.zeros((L,), jnp.float32)
    zerosi = jnp.zeros((L,), jnp.int32)

    def zh(i, _):
        hist[pl.ds(i * L, L)] = zerosi
        return 0
    lax.fori_loop(0, NB // L, zh, 0)

    def zp(i, _):
        pks[pl.ds(i * L, L)] = zerosf
        pki[pl.ds(i * L, L)] = zerosi
        return 0
    lax.fori_loop(0, PK_PAD // L, zp, 0)

    bufs = (buf0, buf1)
    sems = (sem0, sem1)
    cps = [None, None]
    cps[0] = pltpu.async_copy(
        x_hbm.at[pl.ds(wid * ROWS_W, CK_ROWS)], buf0, sem0)
    lanes = lax.iota(jnp.int32, L)
    off = jnp.int32(0)
    for ci in range(N_CK):
        if ci + 1 < N_CK:
            cps[(ci + 1) % 2] = pltpu.async_copy(
                x_hbm.at[pl.ds(wid * ROWS_W + (ci + 1) * CK_ROWS, CK_ROWS)],
                bufs[(ci + 1) % 2], sems[(ci + 1) % 2])
        cps[ci % 2].wait()
        buf = bufs[ci % 2]

        def rbody(r, off, _ci=ci, _buf=buf):
            rowbase = base + (_ci * CK_ROWS) * W + r * W

            def vbody(j, off):
                vs, ms, cs = [], [], []
                for u in range(16):
                    v = _buf[r, pl.ds((j * 16 + u) * L, L)]
                    m = v > 0.5
                    vs.append(v)
                    ms.append(m)
                    cs.append(jnp.sum(m.astype(jnp.int32)))
                for u in range(16):
                    iv = (rowbase + (j * 16 + u) * L) + lanes
                    plsc.store_compressed(pks.at[pl.ds(off, L)], vs[u], mask=ms[u])
                    plsc.store_compressed(pki.at[pl.ds(off, L)], iv, mask=ms[u])
                    off = off + cs[u]
                return off
            off = lax.fori_loop(0, W // L // 16, vbody, off)
            return jnp.minimum(off, PK_CAP)
        off = lax.fori_loop(0, CK_ROWS, rbody, off)
    ones = jnp.ones((L,), jnp.int32)

    def hbody(i, _):
        v = pks[pl.ds(i * L, L)]
        m = v > 0.5
        b = ((v - 0.5) * (2 * NB)).astype(jnp.int32)
        plsc.addupdate_scatter(hist, [b], ones, mask=m)
        return 0
    lax.fori_loop(0, (off + (L - 1)) // L, hbody, 0)
    pltpu.sync_copy(hist, hist_hbm.at[wid])
    pltpu.sync_copy(pks.at[pl.ds(0, PK_CAP)], pks_hbm.at[wid])
    pltpu.sync_copy(pki.at[pl.ds(0, PK_CAP)], pki_hbm.at[wid])


def _sc_scan(masked2d):
    return pl.kernel(
        _scan_body,
        out_type=(
            jax.ShapeDtypeStruct((NW, NB), jnp.int32),
            jax.ShapeDtypeStruct((NW, PK_CAP), jnp.float32),
            jax.ShapeDtypeStruct((NW, PK_CAP), jnp.int32),
        ),
        mesh=_mesh(),
        compiler_params=pltpu.CompilerParams(needs_layout_passes=False),
        scratch_types=[
            pltpu.VMEM((CK_ROWS, W), jnp.float32),
            pltpu.VMEM((CK_ROWS, W), jnp.float32),
            pltpu.VMEM((PK_PAD,), jnp.float32),
            pltpu.VMEM((PK_PAD,), jnp.int32),
            pltpu.VMEM((NB,), jnp.int32),
            pltpu.SemaphoreType.DMA,
            pltpu.SemaphoreType.DMA,
        ],
    )(masked2d)


def _select_body(pks_hbm, pki_hbm, lo_hbm, sc_hbm, ix_hbm,
                 ps, pi, sbuf, ibuf, lvec):
    wid = lax.axis_index("c") * 16 + lax.axis_index("s")
    neg = jnp.full((L,), -jnp.inf, jnp.float32)
    zer = jnp.zeros((L,), jnp.int32)

    def zbody(i, _):
        sbuf[pl.ds(i * L, L)] = neg
        ibuf[pl.ds(i * L, L)] = zer
        return 0
    lax.fori_loop(0, CAP_T // L, zbody, 0)
    pltpu.sync_copy(lo_hbm, lvec)
    lo = lvec[...]
    pltpu.sync_copy(pks_hbm.at[wid], ps)
    pltpu.sync_copy(pki_hbm.at[wid], pi)

    def vbody(i, cnt):
        vs, ms, exs, cs = [], [], [], []
        for u in range(4):
            sl = pl.ds((i * 4 + u) * L, L)
            v = ps[sl]
            m = v >= lo
            mi = m.astype(jnp.int32)
            vs.append(v)
            ms.append(m)
            exs.append(plsc.cumsum(mi) - mi)
            cs.append(jnp.sum(mi))
        for u in range(4):
            pos = cnt + exs[u]
            mg = jnp.logical_and(ms[u], pos < CAP_T)
            posc = jnp.minimum(pos, CAP_T - 1)
            plsc.store_scatter(sbuf, [posc], vs[u], mask=mg)
            plsc.store_scatter(ibuf, [posc], pi[pl.ds((i * 4 + u) * L, L)],
                               mask=mg)
            cnt = cnt + cs[u]
        return cnt
    lax.fori_loop(0, PK_CAP // L // 4, vbody, jnp.int32(0))
    pltpu.sync_copy(sbuf, sc_hbm.at[pl.ds(wid * CAP_T, CAP_T)])
    pltpu.sync_copy(ibuf, ix_hbm.at[pl.ds(wid * CAP_T, CAP_T)])


def _sc_select(pks, pki, lo_vec):
    return pl.kernel(
        _select_body,
        out_type=(
            jax.ShapeDtypeStruct((NW * CAP_T,), jnp.float32),
            jax.ShapeDtypeStruct((NW * CAP_T,), jnp.int32),
        ),
        mesh=_mesh(),
        compiler_params=pltpu.CompilerParams(needs_layout_passes=False),
        scratch_types=[
            pltpu.VMEM((PK_CAP,), jnp.float32),
            pltpu.VMEM((PK_CAP,), jnp.int32),
            pltpu.VMEM((CAP_T,), jnp.float32),
            pltpu.VMEM((CAP_T,), jnp.int32),
            pltpu.VMEM((L,), jnp.float32),
        ],
    )(pks, pki, lo_vec)


def kernel(score_map, k):
    score2d = score_map.reshape(H, W)
    masked = _masked_map(score2d)
    hist32, pks, pki = _sc_scan(masked)
    hist = hist32.sum(axis=0)
    above = jnp.cumsum(hist[::-1])[::-1] >= K
    bstar = jnp.maximum(jnp.sum(above.astype(jnp.int32)) - 1, 0)
    lo = 0.5 + bstar.astype(jnp.float32) * (1.0 / (2 * NB))
    lo_vec = jnp.full((L,), lo, jnp.float32)
    cscores, cidx = _sc_select(pks, pki, lo_vec)
    negs, topi = lax.sort_key_val(-cscores, cidx)
    topv = -negs[:K]
    topi = topi[:K]
    nfin = jnp.sum((topv > 0.5).astype(jnp.int32))
    filler = lax.iota(jnp.int32, K) - nfin
    topi = jnp.where(topv > 0.5, topi, filler)
    topv = jnp.where(topv > 0.5, topv, -jnp.inf)
    topi = topi + jnp.asarray(k - k, dtype=topi.dtype)
    indices = jnp.stack([topi // W, topi % W], axis=-1)
    return (indices[None], topv[None])
